# baseline (speedup 1.0000x reference)
import dataclasses
import functools

import jax
import jax.numpy as jnp
from jax import lax
from jax.experimental import pallas as pl
from jax.experimental.pallas import tpu as pltpu
from jax.experimental.pallas import tpu_sc as plsc

_PAD = 1
_NUM_CORES = 2
_NUM_SUBCORES = 16
_NUM_WORKERS = _NUM_CORES * _NUM_SUBCORES
_LANES = 16
_BLOCK = 512
_ALBUF = 520


@functools.lru_cache(maxsize=None)
def _make_sc_prepare(n_rows, dim, n_table_rows):
    rows_per_w = n_rows // _NUM_WORKERS
    nvec = rows_per_w // _LANES
    half = nvec // 2
    blocks_per_sc = 2 * _NUM_SUBCORES
    tiles_per_row = 8
    chunk = _LANES
    mesh = plsc.VectorSubcoreMesh(core_axis_name="c", subcore_axis_name="s")
    cp = pltpu.CompilerParams()
    if "needs_layout_passes" in pltpu.CompilerParams.__dataclass_fields__:
        cp = dataclasses.replace(cp, needs_layout_passes=False)

    @functools.partial(
        pl.kernel,
        mesh=mesh,
        compiler_params=cp,
        out_type=(
            jax.ShapeDtypeStruct((n_rows, dim), jnp.float32),
            jax.ShapeDtypeStruct((_NUM_WORKERS, _LANES), jnp.int32),
        ),
        scratch_types=[
            pltpu.VMEM((nvec, _LANES), jnp.int32),
            pltpu.VMEM((nvec, _LANES), jnp.int32),
            pltpu.VMEM((_LANES,), jnp.int32),
            pltpu.VMEM((_LANES,), jnp.int32),
            pltpu.VMEM((_NUM_SUBCORES, _LANES), jnp.int32),
            pltpu.VMEM((_NUM_SUBCORES, _LANES), jnp.int32),
            pltpu.VMEM((chunk,), jnp.int32),
            pltpu.VMEM((chunk, dim), jnp.float32),
            pltpu.VMEM_SHARED((_NUM_SUBCORES, _LANES), jnp.int32),
            pltpu.VMEM_SHARED((_NUM_SUBCORES, _LANES), jnp.int32),
            pltpu.VMEM_SHARED((_NUM_SUBCORES, nvec, _LANES), jnp.int32),
            pltpu.SMEM((blocks_per_sc,), jnp.int32),
            pltpu.SMEM((1,), jnp.int32),
            pltpu.SemaphoreType.DMA,
        ],
    )
    def sc_prepare(tok_hbm, table_hbm, fix_hbm, aux_hbm, tok_v, idx_v,
                   carry_v, auxvec_v, tot_v, auxloc_v, fidx_v, frows_v,
                   sh_tot, sh_aux, sh_pos, list_ref, cnt_ref, sem):
        cid = lax.axis_index("c")
        sid = lax.axis_index("s")
        wid = cid * _NUM_SUBCORES + sid

        pltpu.sync_copy(tok_hbm.at[wid], tok_v)
        carry_v[...] = jnp.zeros((_LANES,), jnp.int32)

        def scan_step(i):
            t = tok_v[i]
            m = jnp.where(t != _PAD, 1, 0).astype(jnp.int32)
            idx_v[i] = plsc.cumsum(m) + carry_v[...]
            carry_v[...] = carry_v[...] + jnp.sum(m)

        @pl.loop(0, half)
        def _(i):
            scan_step(i)

        mid = jnp.max(carry_v[...])

        @pl.loop(half, nvec)
        def _(i):
            scan_step(i)

        total = jnp.max(carry_v[...])

        pltpu.sync_copy(carry_v, sh_tot.at[sid])
        plsc.subcore_barrier()
        pltpu.sync_copy(sh_tot, tot_v)
        group_start = (sid // tiles_per_row) * tiles_per_row
        prefix = jnp.int32(0)
        for j in range(_NUM_SUBCORES):
            tj = jnp.max(tot_v[j])
            take = jnp.logical_and(j >= group_start, j < sid)
            prefix = prefix + jnp.where(take, tj, 0)

        @pl.loop(0, nvec)
        def _(i):
            t = tok_v[i]
            m = jnp.where(t != _PAD, 1, 0).astype(jnp.int32)
            idx_v[i] = (idx_v[i] + prefix) * m + _PAD

        lanes = lax.iota(jnp.int32, _LANES)
        s0 = prefix + 2
        s1 = prefix + mid + 2
        lim = jnp.int32(n_table_rows - _ALBUF)
        d0 = jnp.where(
            jnp.logical_and(mid == jnp.int32(_BLOCK), s0 <= lim), 0, 1)
        d1 = jnp.where(
            jnp.logical_and(total - mid == jnp.int32(_BLOCK), s1 <= lim), 0, 1)
        auxvec_v[...] = (
            jnp.where(lanes == 0, s0, 0)
            + jnp.where(lanes == 1, s1, 0)
            + jnp.where(lanes == 2, d0, 0)
            + jnp.where(lanes == 3, d1, 0)
        )
        pltpu.sync_copy(auxvec_v, aux_hbm.at[wid])
        pltpu.sync_copy(auxvec_v, sh_aux.at[sid])
        pltpu.sync_copy(idx_v, sh_pos.at[sid])
        plsc.subcore_barrier()

        pltpu.sync_copy(sh_aux, auxloc_v)
        cnt_ref[0] = 0

        @pl.loop(0, blocks_per_sc)
        def _(gl):
            row = auxloc_v[gl // 2]
            d = jnp.max(jnp.where(lanes == 2 + (gl % 2), row, 0))
            c = cnt_ref[0]

            @pl.when(d != 0)
            def _():
                list_ref[c] = gl
                cnt_ref[0] = c + 1

        chunks_per_block = _BLOCK // chunk

        @pl.loop(0, cnt_ref[0])
        def _(n):
            gl = list_ref[n]
            owner = gl // 2
            r = gl % 2
            for k in range(2):
                ck = r * chunks_per_block + sid * 2 + k
                pltpu.sync_copy(sh_pos.at[owner, ck], fidx_v)
                pltpu.async_copy(table_hbm.at[fidx_v], frows_v, sem).wait()
                flat = (cid * _NUM_SUBCORES + owner) * rows_per_w + ck * chunk
                pltpu.sync_copy(frows_v, fix_hbm.at[pl.ds(flat, chunk)])

    return sc_prepare


@functools.lru_cache(maxsize=None)
def _make_tc_assemble(n_rows, dim):
    n_blocks = n_rows // _BLOCK
    n_steps = n_blocks // 2

    def tc_body(aux_ref, table_hbm, fix_hbm, out_ref, a0, a1, s0, s1):
        i = pl.program_id(0)
        abufs = (a0, a1)
        sems = (s0, s1)

        def meta(blk):
            t = blk // 2
            r = blk % 2
            return aux_ref[t, r], aux_ref[t, 2 + r]

        def issue(blk, b):
            start, dirty = meta(blk)

            @pl.when(dirty == 0)
            def _():
                al = pl.multiple_of((start // 8) * 8, 8)
                pltpu.make_async_copy(
                    table_hbm.at[pl.ds(al, _ALBUF)], abufs[b], sems[b]
                ).start()

            @pl.when(dirty != 0)
            def _():
                pltpu.make_async_copy(
                    fix_hbm.at[pl.ds(blk * _BLOCK, _BLOCK)],
                    abufs[b].at[pl.ds(0, _BLOCK)],
                    sems[b],
                ).start()

        def wait(blk, b):
            _, dirty = meta(blk)

            @pl.when(dirty == 0)
            def _():
                pltpu.make_async_copy(
                    table_hbm.at[pl.ds(0, _ALBUF)], abufs[b], sems[b]
                ).wait()

            @pl.when(dirty != 0)
            def _():
                pltpu.make_async_copy(
                    fix_hbm.at[pl.ds(0, _BLOCK)],
                    abufs[b].at[pl.ds(0, _BLOCK)],
                    sems[b],
                ).wait()

        def consume(blk, b, half):
            start, dirty = meta(blk)
            wait(blk, b)
            delta = jnp.where(dirty == 0, start % 8, 0)
            for k in range(8):
                @pl.when(delta == k)
                def _():
                    out_ref[pl.ds(half * _BLOCK, _BLOCK), :] = (
                        abufs[b][pl.ds(k, _BLOCK), :]
                    )

        @pl.when(i == 0)
        def _():
            issue(0, 0)
            issue(1, 1)

        consume(2 * i, 0, 0)

        @pl.when(i + 1 < n_steps)
        def _():
            issue(2 * i + 2, 0)

        consume(2 * i + 1, 1, 1)

        @pl.when(i + 1 < n_steps)
        def _():
            issue(2 * i + 3, 1)

    return pl.pallas_call(
        tc_body,
        grid=(n_steps,),
        in_specs=[
            pl.BlockSpec(memory_space=pltpu.SMEM),
            pl.BlockSpec(memory_space=pl.ANY),
            pl.BlockSpec(memory_space=pl.ANY),
        ],
        out_specs=pl.BlockSpec((2 * _BLOCK, dim), lambda i: (i, 0)),
        out_shape=jax.ShapeDtypeStruct((n_rows, dim), jnp.float32),
        scratch_shapes=[
            pltpu.VMEM((_ALBUF, dim), jnp.float32),
            pltpu.VMEM((_ALBUF, dim), jnp.float32),
            pltpu.SemaphoreType.DMA,
            pltpu.SemaphoreType.DMA,
        ],
    )


def kernel(tokens, table):
    batch, seq = tokens.shape
    n_rows = batch * seq
    dim = table.shape[1]

    tok3 = tokens.reshape(_NUM_WORKERS, (n_rows // _NUM_WORKERS) // _LANES, _LANES)
    fix, aux = _make_sc_prepare(n_rows, dim, table.shape[0])(tok3, table)
    out = _make_tc_assemble(n_rows, dim)(aux, table, fix)
    return out.reshape(batch, seq, dim)


# speedup vs baseline: 2.1413x; 2.1413x over previous
import dataclasses
import functools

import jax
import jax.numpy as jnp
from jax import lax
from jax.experimental import pallas as pl
from jax.experimental.pallas import tpu as pltpu
from jax.experimental.pallas import tpu_sc as plsc

_PAD = 1
_NUM_CORES = 2
_NUM_SUBCORES = 16
_NUM_WORKERS = _NUM_CORES * _NUM_SUBCORES
_LANES = 16


@functools.lru_cache(maxsize=None)
def _make_embed(n_rows, dim, chunk):
    rows_per_w = n_rows // _NUM_WORKERS
    nchunk = rows_per_w // chunk
    nvec = rows_per_w // _LANES
    tiles_per_row = 8
    mesh = plsc.VectorSubcoreMesh(core_axis_name="c", subcore_axis_name="s")
    cp = pltpu.CompilerParams()
    if "needs_layout_passes" in pltpu.CompilerParams.__dataclass_fields__:
        cp = dataclasses.replace(cp, needs_layout_passes=False)

    @functools.partial(
        pl.kernel,
        mesh=mesh,
        compiler_params=cp,
        out_type=jax.ShapeDtypeStruct((n_rows, dim), jnp.float32),
        scratch_types=[
            pltpu.VMEM((nvec, _LANES), jnp.int32),
            pltpu.VMEM((nchunk, chunk), jnp.int32),
            pltpu.VMEM((_LANES,), jnp.int32),
            pltpu.VMEM((_NUM_SUBCORES, _LANES), jnp.int32),
            pltpu.VMEM_SHARED((_NUM_SUBCORES, _LANES), jnp.int32),
            pltpu.VMEM((chunk, dim), jnp.float32),
            pltpu.VMEM((chunk, dim), jnp.float32),
            pltpu.VMEM((chunk, dim), jnp.float32),
            pltpu.VMEM((chunk, dim), jnp.float32),
            pltpu.VMEM((chunk, dim), jnp.float32),
            pltpu.VMEM((chunk, dim), jnp.float32),
            pltpu.SemaphoreType.DMA,
            pltpu.SemaphoreType.DMA,
            pltpu.SemaphoreType.DMA,
            pltpu.SemaphoreType.DMA,
            pltpu.SemaphoreType.DMA,
            pltpu.SemaphoreType.DMA,
            pltpu.SemaphoreType.DMA,
            pltpu.SemaphoreType.DMA,
            pltpu.SemaphoreType.DMA,
            pltpu.SemaphoreType.DMA,
            pltpu.SemaphoreType.DMA,
            pltpu.SemaphoreType.DMA,
        ],
    )
    def embed_kernel(tok_hbm, table_hbm, out_hbm, tok_v, idx_v, carry_v,
                     tot_v, shared_v, b0, b1, b2, b3, b4, b5,
                     g0, g1, g2, g3, g4, g5, w0, w1, w2, w3, w4, w5):
        cid = lax.axis_index("c")
        sid = lax.axis_index("s")
        wid = cid * _NUM_SUBCORES + sid
        base = wid * rows_per_w

        pltpu.sync_copy(tok_hbm.at[wid], tok_v)
        carry_v[...] = jnp.zeros((_LANES,), jnp.int32)

        @pl.loop(0, nvec)
        def _(i):
            t = tok_v[i]
            m = jnp.where(t != _PAD, 1, 0).astype(jnp.int32)
            raw = plsc.cumsum(m) + carry_v[...]
            idx_v[i] = raw
            carry_v[...] = carry_v[...] + jnp.sum(m)

        pltpu.sync_copy(carry_v, shared_v.at[sid])
        plsc.subcore_barrier()
        pltpu.sync_copy(shared_v, tot_v)
        group_start = (sid // tiles_per_row) * tiles_per_row
        prefix = jnp.int32(0)
        for j in range(_NUM_SUBCORES):
            tj = jnp.max(tot_v[j])
            take = jnp.logical_and(j >= group_start, j < sid)
            prefix = prefix + jnp.where(take, tj, 0)

        @pl.loop(0, nvec)
        def _(i):
            t = tok_v[i]
            m = jnp.where(t != _PAD, 1, 0).astype(jnp.int32)
            idx_v[i] = (idx_v[i] + prefix) * m + _PAD

        bufs = (b0, b1, b2, b3, b4, b5)
        gsems = (g0, g1, g2, g3, g4, g5)
        wsems = (w0, w1, w2, w3, w4, w5)

        def start_g(g, b):
            pltpu.make_async_copy(table_hbm.at[idx_v.at[g]], bufs[b], gsems[b]).start()

        def wait_g(b):
            pltpu.make_async_copy(table_hbm.at[idx_v.at[0]], bufs[b], gsems[b]).wait()

        def start_w(g, b):
            pltpu.make_async_copy(
                bufs[b], out_hbm.at[pl.ds(base + g * chunk, chunk)], wsems[b]
            ).start()

        def wait_w(b):
            pltpu.make_async_copy(
                bufs[b], out_hbm.at[pl.ds(base, chunk)], wsems[b]
            ).wait()

        start_g(0, 0)
        start_g(1, 1)
        start_g(2, 2)
        wait_g(0)
        start_w(0, 0)
        start_g(3, 3)
        wait_g(1)
        start_w(1, 1)
        start_g(4, 4)
        wait_g(2)
        start_w(2, 2)
        start_g(5, 5)
        wait_g(3)
        start_w(3, 3)
        wait_w(0)
        start_g(6, 0)
        wait_g(4)
        start_w(4, 4)
        wait_w(1)
        start_g(7, 1)
        wait_g(5)
        start_w(5, 5)
        wait_w(2)
        start_g(8, 2)

        @pl.loop(6, ((nchunk - 4) // 6) * 6, step=6)
        def _(c0):
            for j in range(6):
                wait_g(j)
                start_w(c0 + j, j)
                wait_w((j + 3) % 6)
                start_g(c0 + j + 3, (j + 3) % 6)

        tail0 = ((nchunk - 4) // 6) * 6
        for c in range(tail0, nchunk):
            b = c % 6
            wait_g(b)
            start_w(c, b)
            if c + 3 < nchunk:
                wait_w((b + 3) % 6)
                start_g(c + 3, (b + 3) % 6)
        for c in range(nchunk - 6, nchunk):
            wait_w(c % 6)

    return embed_kernel


def kernel(tokens, table):
    batch, seq = tokens.shape
    n_rows = batch * seq
    dim = table.shape[1]
    chunk = _LANES

    tok3 = tokens.reshape(_NUM_WORKERS, (n_rows // _NUM_WORKERS) // _LANES, _LANES)
    out = _make_embed(n_rows, dim, chunk)(tok3, table)
    return out.reshape(batch, seq, dim)
